# baseline (speedup 1.0000x reference)
import functools

import jax
import jax.numpy as jnp
from jax import lax
from jax.experimental import pallas as pl
from jax.experimental.pallas import tpu as pltpu
from jax.experimental.pallas import tpu_sc as plsc

_NUM_PROMPTS = 12
_PROMPT_LEN = 20
_HIDDEN = 4096
_BATCH = 1024

_NC = 2
_NS = 16
_NW = _NC * _NS

_BPW = _BATCH // _NW
_PIECES = ((0, 16), (16, 4))


def _sc_gather(ridx, table2d):
    mesh = plsc.VectorSubcoreMesh(core_axis_name="c", subcore_axis_name="s")

    @functools.partial(
        pl.kernel,
        mesh=mesh,
        compiler_params=pltpu.CompilerParams(use_tc_tiling_on_sc=True),
        out_type=jax.ShapeDtypeStruct((_PROMPT_LEN, _BATCH, _HIDDEN),
                                      jnp.float32),
        scratch_types=(
            [pltpu.VMEM((_BPW, _PROMPT_LEN), jnp.int32)]
            + [pltpu.VMEM((ln, _HIDDEN), jnp.float32) for _, ln in _PIECES]
            + [pltpu.SemaphoreType.DMA] * 4
        ),
    )
    def k(ridx_hbm, table_hbm, out_hbm, ridx_v, buf0, buf1,
          g0, g1, s0, s1):
        bufs = (buf0, buf1)
        gsems = (g0, g1)
        ssems = (s0, s1)
        wid = lax.axis_index("s") * _NC + lax.axis_index("c")
        base = wid * _BPW

        pltpu.sync_copy(ridx_hbm.at[wid], ridx_v)

        def body(j, carry):
            bb = base + j
            for s, (so, ln) in enumerate(_PIECES):
                @pl.when(j > 0)
                def _wait_prev_store():
                    pltpu.make_async_copy(
                        bufs[s],
                        out_hbm.at[pl.ds(so, ln), bb - 1, :],
                        ssems[s],
                    ).wait()

                pltpu.make_async_copy(
                    table_hbm.at[ridx_v.at[j, pl.ds(so, ln)]],
                    bufs[s],
                    gsems[s],
                ).start()
            for s, (so, ln) in enumerate(_PIECES):
                pltpu.make_async_copy(
                    table_hbm.at[ridx_v.at[j, pl.ds(so, ln)]],
                    bufs[s],
                    gsems[s],
                ).wait()
                pltpu.make_async_copy(
                    bufs[s],
                    out_hbm.at[pl.ds(so, ln), bb, :],
                    ssems[s],
                ).start()
            return carry

        lax.fori_loop(0, _BPW, body, 0)

        for s, (so, ln) in enumerate(_PIECES):
            pltpu.make_async_copy(
                bufs[s],
                out_hbm.at[pl.ds(so, ln), base + _BPW - 1, :],
                ssems[s],
            ).wait()

    return k(ridx, table2d)


def kernel(cultural_context, cultural_prompts):
    idx = cultural_context.astype(jnp.int32)
    ridx = (idx.reshape(_NW, _BPW, 1) * _PROMPT_LEN
            + jnp.arange(_PROMPT_LEN, dtype=jnp.int32)[None, None, :])
    table2d = cultural_prompts.reshape(_NUM_PROMPTS * _PROMPT_LEN, _HIDDEN)
    out = _sc_gather(ridx, table2d)
    return jnp.transpose(out, (1, 0, 2))


# speedup vs baseline: 1.0060x; 1.0060x over previous
import functools

import jax
import jax.numpy as jnp
from jax import lax
from jax.experimental import pallas as pl
from jax.experimental.pallas import tpu as pltpu
from jax.experimental.pallas import tpu_sc as plsc

_NUM_PROMPTS = 12
_PROMPT_LEN = 20
_HIDDEN = 4096
_BATCH = 1024

_NC = 2
_NS = 16
_NW = _NC * _NS

_K = 8
_G = 160
_M = _G // _NW
_S = _G * _K
_PIECES = ((0, 16), (16, 4))


def _sc_gather(ridx, sidx, table2d):
    mesh = plsc.VectorSubcoreMesh(core_axis_name="c", subcore_axis_name="s")

    @functools.partial(
        pl.kernel,
        mesh=mesh,
        compiler_params=pltpu.CompilerParams(use_tc_tiling_on_sc=True),
        out_type=jax.ShapeDtypeStruct((_PROMPT_LEN * _BATCH, _HIDDEN),
                                      jnp.float32),
        scratch_types=(
            [pltpu.VMEM((_M, _PROMPT_LEN), jnp.int32),
             pltpu.VMEM((_M * _K, _PROMPT_LEN), jnp.int32)]
            + [pltpu.VMEM((ln, _HIDDEN), jnp.float32) for _, ln in _PIECES]
            + [pltpu.SemaphoreType.DMA] * 4
        ),
    )
    def k(ridx_hbm, sidx_hbm, table_hbm, out_hbm,
          ridx_v, sidx_v, buf0, buf1, g0, g1, s0, s1):
        bufs = (buf0, buf1)
        gsems = (g0, g1)
        ssems = (s0, s1)
        wid = lax.axis_index("s") * _NC + lax.axis_index("c")

        pltpu.sync_copy(ridx_hbm.at[wid], ridx_v)
        pltpu.sync_copy(sidx_hbm.at[wid], sidx_v)

        def drain_group(_):
            for s, (so, ln) in enumerate(_PIECES):
                for _e in range(_K):
                    pltpu.make_async_copy(
                        bufs[s],
                        out_hbm.at[sidx_v.at[0, pl.ds(so, ln)]],
                        ssems[s],
                    ).wait()

        def group(g, carry):
            @pl.when(g > 0)
            def _():
                drain_group(None)

            for s, (so, ln) in enumerate(_PIECES):
                pltpu.make_async_copy(
                    table_hbm.at[ridx_v.at[g, pl.ds(so, ln)]],
                    bufs[s],
                    gsems[s],
                ).start()
            for s, (so, ln) in enumerate(_PIECES):
                pltpu.make_async_copy(
                    table_hbm.at[ridx_v.at[g, pl.ds(so, ln)]],
                    bufs[s],
                    gsems[s],
                ).wait()

            for e in range(_K):
                for s, (so, ln) in enumerate(_PIECES):
                    pltpu.make_async_copy(
                        bufs[s],
                        out_hbm.at[sidx_v.at[g * _K + e, pl.ds(so, ln)]],
                        ssems[s],
                    ).start()
            return carry

        lax.fori_loop(0, _M, group, 0)
        drain_group(None)

    return k(ridx, sidx, table2d)


def kernel(cultural_context, cultural_prompts):
    idx = cultural_context.astype(jnp.int32)
    order = jnp.argsort(idx).astype(jnp.int32)
    n = jnp.bincount(idx, length=_NUM_PROMPTS).astype(jnp.int32)
    m = ((n + _K - 1) // _K) * _K
    moff = jnp.cumsum(m)
    noff = jnp.cumsum(n) - n
    total = moff[-1]
    s = jnp.arange(_S, dtype=jnp.int32) % total
    p = jnp.searchsorted(moff, s, side="right").astype(jnp.int32)
    j = s - (moff[p] - m[p])
    e = noff[p] + jnp.minimum(j, n[p] - 1)
    dest = order[e]
    t = jnp.arange(_PROMPT_LEN, dtype=jnp.int32)
    sidx = (t[None, :] * _BATCH + dest[:, None]).reshape(
        _NW, _M * _K, _PROMPT_LEN)
    gp = p.reshape(_G, _K)[:, 0]
    ridx = (gp[:, None] * _PROMPT_LEN + t[None, :]).reshape(
        _NW, _M, _PROMPT_LEN)
    table2d = cultural_prompts.reshape(_NUM_PROMPTS * _PROMPT_LEN, _HIDDEN)
    out = _sc_gather(ridx, sidx, table2d)
    out = out.reshape(_PROMPT_LEN, _BATCH, _HIDDEN)
    return jnp.transpose(out, (1, 0, 2))
